# baseline (speedup 1.0000x reference)
import functools

import jax
import jax.numpy as jnp
from jax import lax
from jax.experimental import pallas as pl
from jax.experimental.pallas import tpu as pltpu
from jax.experimental.pallas import tpu_sc as plsc

N = 10000
E = 320000
D = 128
NC, NS, LANES = 2, 16, 16
NW = NC * NS
CHUNK = 64
NBUF = 4
NCH_TILE = 160
NCHUNKS_P = NW * NCH_TILE + NBUF
E_PAD = NCHUNKS_P * CHUNK
N_PAD = 10240
ROWS_PER_TILE = N_PAD // NS
ZBLK = 64

_mesh = plsc.VectorSubcoreMesh(core_axis_name="c", subcore_axis_name="s")


def _wid():
    c = lax.axis_index("c")
    s = lax.axis_index("s")
    return c, s, s * NC + c


@functools.partial(
    pl.kernel,
    out_type=jax.ShapeDtypeStruct((NC, N_PAD, LANES), jnp.float32),
    mesh=_mesh,
    scratch_types=[
        pltpu.VMEM_SHARED((N_PAD, LANES), jnp.float32),
        pltpu.VMEM((NBUF, 2, CHUNK), jnp.int32),
        pltpu.VMEM((NBUF, CHUNK, LANES), jnp.float32),
        pltpu.SemaphoreType.DMA((NBUF,)),
        pltpu.SemaphoreType.DMA((NBUF,)),
    ],
)
def _sc_degree(idx2_hbm, ewb_hbm, out_hbm, acc_sh, idx_v, ewb_v, sem_io, sem_s):
    c, s, wid = _wid()
    row0 = s * ROWS_PER_TILE
    start = wid * NCH_TILE

    def prefetch(b, q):
        pltpu.async_copy(idx2_hbm.at[q], idx_v.at[b], sem_io.at[b])
        pltpu.async_copy(ewb_hbm.at[q], ewb_v.at[b], sem_io.at[b])

    def wait_prefetch(b, q):
        pltpu.make_async_copy(idx2_hbm.at[q], idx_v.at[b], sem_io.at[b]).wait()
        pltpu.make_async_copy(ewb_hbm.at[q], ewb_v.at[b], sem_io.at[b]).wait()

    def scatter(b):
        pltpu.async_copy(ewb_v.at[b], acc_sh.at[idx_v.at[b, 1]], sem_s.at[b],
                         add=True)

    def wait_scatter(b):
        pltpu.make_async_copy(ewb_v.at[b], acc_sh.at[idx_v.at[b, 1]],
                              sem_s.at[b]).wait()

    @pl.loop(0, CHUNK)
    def _(r):
        ewb_v[0, r, :] = jnp.zeros((LANES,), jnp.float32)

    @pl.loop(0, ROWS_PER_TILE // ZBLK)
    def _(i):
        pltpu.sync_copy(ewb_v.at[0],
                        acc_sh.at[pl.ds(row0 + i * ZBLK, ZBLK)])

    plsc.subcore_barrier()

    prefetch(0, start)
    prefetch(1, start + 1)
    for t in range(2, NBUF + 2):
        b = t - 2
        wait_prefetch(b, start + b)
        scatter(b)
        if b >= 2:
            wait_scatter(b - 2)
        prefetch((b + 2) % NBUF, start + b + 2)

    @pl.loop(1, NCH_TILE // NBUF)
    def _(i):
        base = start + i * NBUF
        for b in range(NBUF):
            q = base + b
            wait_prefetch(b, q)
            scatter(b)
            m = (b + 2) % NBUF
            wait_scatter(m)
            prefetch(m, q + 2)

    for b in range(NBUF):
        if b < 2:
            wait_prefetch(b, start + NCH_TILE + b)
        else:
            wait_scatter(b)

    plsc.subcore_barrier()

    @pl.loop(0, ROWS_PER_TILE // ZBLK)
    def _(i):
        r = row0 + i * ZBLK
        pltpu.sync_copy(acc_sh.at[pl.ds(r, ZBLK)], ewb_v.at[0])
        pltpu.sync_copy(ewb_v.at[0], out_hbm.at[c, pl.ds(r, ZBLK)])


@functools.partial(
    pl.kernel,
    out_type=jax.ShapeDtypeStruct((NC, N_PAD, D), jnp.float32),
    mesh=_mesh,
    scratch_types=[
        pltpu.VMEM_SHARED((N_PAD, D), jnp.float32),
        pltpu.VMEM((NBUF, 2, CHUNK), jnp.int32),
        pltpu.VMEM((NBUF, CHUNK * LANES), jnp.float32),
        pltpu.VMEM((NBUF, CHUNK, D), jnp.float32),
        pltpu.SemaphoreType.DMA((NBUF,)),
        pltpu.SemaphoreType.DMA((NBUF,)),
        pltpu.SemaphoreType.DMA((NBUF,)),
    ],
)
def _sc_aggregate(idx2_hbm, ewb_hbm, hs_hbm, out_hbm,
                  acc_sh, idx_v, ewb_v, rows_v, sem_io, sem_g, sem_s):
    c, s, wid = _wid()
    row0 = s * ROWS_PER_TILE
    start = wid * NCH_TILE

    def prefetch(b, q):
        pltpu.async_copy(idx2_hbm.at[q], idx_v.at[b], sem_io.at[b])
        pltpu.async_copy(ewb_hbm.at[q], ewb_v.at[b], sem_io.at[b])

    def wait_prefetch(b, q):
        pltpu.make_async_copy(idx2_hbm.at[q], idx_v.at[b], sem_io.at[b]).wait()
        pltpu.make_async_copy(ewb_hbm.at[q], ewb_v.at[b], sem_io.at[b]).wait()

    def launch_gather(b, q):
        wait_prefetch(b, q)
        pltpu.async_copy(hs_hbm.at[idx_v.at[b, 0]], rows_v.at[b], sem_g.at[b])

    def process(b):
        pltpu.make_async_copy(hs_hbm.at[idx_v.at[b, 0]], rows_v.at[b],
                              sem_g.at[b]).wait()

        @pl.loop(0, CHUNK)
        def _(e):
            w = ewb_v[b, pl.ds(e * LANES, LANES)]
            for k in range(D // LANES):
                sl = pl.ds(k * LANES, LANES)
                rows_v[b, e, sl] = rows_v[b, e, sl] * w

        pltpu.async_copy(rows_v.at[b], acc_sh.at[idx_v.at[b, 1]], sem_s.at[b],
                         add=True)

    def wait_scatter(b):
        pltpu.make_async_copy(rows_v.at[b], acc_sh.at[idx_v.at[b, 1]],
                              sem_s.at[b]).wait()

    @pl.loop(0, CHUNK)
    def _(r):
        for k in range(D // LANES):
            rows_v[0, r, pl.ds(k * LANES, LANES)] = jnp.zeros((LANES,),
                                                              jnp.float32)

    @pl.loop(0, ROWS_PER_TILE // ZBLK)
    def _(i):
        pltpu.sync_copy(rows_v.at[0],
                        acc_sh.at[pl.ds(row0 + i * ZBLK, ZBLK)])

    plsc.subcore_barrier()

    prefetch(0, start)
    prefetch(1, start + 1)
    prefetch(2, start + 2)
    launch_gather(0, start)
    launch_gather(1, start + 1)
    for b in range(NBUF):
        process(b)
        mb = (b + 3) % NBUF
        if b >= 1:
            wait_scatter(mb)
        prefetch(mb, start + b + 3)
        launch_gather((b + 2) % NBUF, start + b + 2)

    @pl.loop(1, NCH_TILE // NBUF)
    def _(i):
        base = start + i * NBUF
        for b in range(NBUF):
            q = base + b
            process(b)
            mb = (b + 3) % NBUF
            wait_scatter(mb)
            prefetch(mb, q + 3)
            launch_gather((b + 2) % NBUF, q + 2)

    wait_scatter(3)
    pltpu.make_async_copy(hs_hbm.at[idx_v.at[0, 0]], rows_v.at[0],
                          sem_g.at[0]).wait()
    pltpu.make_async_copy(hs_hbm.at[idx_v.at[1, 0]], rows_v.at[1],
                          sem_g.at[1]).wait()
    wait_prefetch(2, start + NCH_TILE + 2)

    plsc.subcore_barrier()

    @pl.loop(0, ROWS_PER_TILE // ZBLK)
    def _(i):
        r = row0 + i * ZBLK
        pltpu.sync_copy(acc_sh.at[pl.ds(r, ZBLK)], rows_v.at[0])
        pltpu.sync_copy(rows_v.at[0], out_hbm.at[c, pl.ds(r, ZBLK)])


MBLK = 1000
EBLK = 20000


def _ewb_body(ew_ref, out_ref):
    out_ref[...] = jnp.broadcast_to(ew_ref[...], (EBLK, LANES))


def _ew_broadcast(ew):
    return pl.pallas_call(
        _ewb_body,
        grid=(E // EBLK,),
        in_specs=[pl.BlockSpec((EBLK, 1), lambda i: (i, 0))],
        out_specs=pl.BlockSpec((EBLK, LANES), lambda i: (i, 0)),
        out_shape=jax.ShapeDtypeStruct((E, LANES), jnp.float32),
    )(ew)


def _mm_body(x_ref, w_ref, out_ref):
    out_ref[...] = jnp.dot(x_ref[...], w_ref[...],
                           preferred_element_type=jnp.float32)


def _matmul(x, w):
    return pl.pallas_call(
        _mm_body,
        grid=(N // MBLK,),
        in_specs=[pl.BlockSpec((MBLK, D), lambda i: (i, 0)),
                  pl.BlockSpec((D, D), lambda i: (0, 0))],
        out_specs=pl.BlockSpec((MBLK, D), lambda i: (i, 0)),
        out_shape=jax.ShapeDtypeStruct((N, D), jnp.float32),
    )(x, w)


def _scale_body(dgp_ref, h_ref, hs_ref, dinv_ref):
    deg = dgp_ref[0, :, 0:1] + dgp_ref[1, :, 0:1] + 1.0
    dinv = lax.rsqrt(deg)
    dinv_ref[...] = dinv
    hs_ref[...] = h_ref[...] * dinv


def _scale(dgp, h):
    return pl.pallas_call(
        _scale_body,
        grid=(N // MBLK,),
        in_specs=[pl.BlockSpec((NC, MBLK, LANES), lambda i: (0, i, 0)),
                  pl.BlockSpec((MBLK, D), lambda i: (i, 0))],
        out_specs=[pl.BlockSpec((MBLK, D), lambda i: (i, 0)),
                   pl.BlockSpec((MBLK, 1), lambda i: (i, 0))],
        out_shape=[jax.ShapeDtypeStruct((N, D), jnp.float32),
                   jax.ShapeDtypeStruct((N, 1), jnp.float32)],
    )(dgp, h)


def _elu(t):
    return jnp.where(t > 0.0, t, jnp.exp(t) - 1.0)


def _mid_body(p_ref, hs_ref, dinv_ref, w_ref, b_ref, out_ref):
    t = (p_ref[0] + p_ref[1] + hs_ref[...]) * dinv_ref[...] + b_ref[...]
    a = _elu(t)
    out_ref[...] = jnp.dot(a, w_ref[...],
                           preferred_element_type=jnp.float32) * dinv_ref[...]


def _mid(p, hs, dinv, w, b):
    return pl.pallas_call(
        _mid_body,
        grid=(N // MBLK,),
        in_specs=[pl.BlockSpec((NC, MBLK, D), lambda i: (0, i, 0)),
                  pl.BlockSpec((MBLK, D), lambda i: (i, 0)),
                  pl.BlockSpec((MBLK, 1), lambda i: (i, 0)),
                  pl.BlockSpec((D, D), lambda i: (0, 0)),
                  pl.BlockSpec((1, D), lambda i: (0, 0))],
        out_specs=pl.BlockSpec((MBLK, D), lambda i: (i, 0)),
        out_shape=jax.ShapeDtypeStruct((N, D), jnp.float32),
    )(p, hs, dinv, w, b)


def _final_body(q_ref, hs_ref, dinv_ref, b_ref, wl_ref, bl_ref, out_ref):
    t = (q_ref[0] + q_ref[1] + hs_ref[...]) * dinv_ref[...] + b_ref[...]
    a = _elu(t)
    out_ref[...] = jnp.sum(a * wl_ref[...], axis=1, keepdims=True) + bl_ref[...]


def _final(q, hs, dinv, b, wlin_t, blin):
    return pl.pallas_call(
        _final_body,
        grid=(N // MBLK,),
        in_specs=[pl.BlockSpec((NC, MBLK, D), lambda i: (0, i, 0)),
                  pl.BlockSpec((MBLK, D), lambda i: (i, 0)),
                  pl.BlockSpec((MBLK, 1), lambda i: (i, 0)),
                  pl.BlockSpec((1, D), lambda i: (0, 0)),
                  pl.BlockSpec((1, D), lambda i: (0, 0)),
                  pl.BlockSpec((1, 1), lambda i: (0, 0))],
        out_specs=pl.BlockSpec((MBLK, 1), lambda i: (i, 0)),
        out_shape=jax.ShapeDtypeStruct((N, 1), jnp.float32),
    )(q, hs, dinv, b, wlin_t, blin)


def kernel(x, edge_index, weights_matrix, W1, b1, W2, b2, Wlin, blin):
    pad = E_PAD - E
    src_p = jnp.concatenate([edge_index[0],
                             jnp.zeros((pad,), jnp.int32)])
    dst_p = jnp.concatenate([edge_index[1],
                             jnp.full((pad,), N, jnp.int32)])
    idx2 = (jnp.stack([src_p, dst_p], axis=0)
            .reshape(2, NCHUNKS_P, CHUNK).transpose(1, 0, 2))

    ewb = _ew_broadcast(weights_matrix.reshape(E, 1))
    ewb = jnp.concatenate([ewb, jnp.zeros((pad, LANES), jnp.float32)])
    ewb3 = ewb.reshape(NCHUNKS_P, CHUNK, LANES)
    ewbf = ewb.reshape(NCHUNKS_P, CHUNK * LANES)

    dgp = _sc_degree(idx2, ewb3)
    h1 = _matmul(x, W1)

    hs1, dinv = _scale(dgp, h1)
    p = _sc_aggregate(idx2, ewbf, hs1)
    hs2 = _mid(p, hs1, dinv, W2, b1.reshape(1, D))
    q = _sc_aggregate(idx2, ewbf, hs2)
    out = _final(q, hs2, dinv, b2.reshape(1, D), Wlin.reshape(1, D),
                 blin.reshape(1, 1))
    return out.reshape(N)


# speedup vs baseline: 1.4819x; 1.4819x over previous
import functools

import jax
import jax.numpy as jnp
from jax import lax
from jax.experimental import pallas as pl
from jax.experimental.pallas import tpu as pltpu
from jax.experimental.pallas import tpu_sc as plsc

N = 10000
E = 320000
D = 128
DH = D // 2
NC, NS, LANES = 2, 16, 16
NW = NC * NS
CHUNK = 64
NBUF = 4
NCHT_D = 160
NCHT_A = 320
NCHUNKS_P = NW * NCHT_D + NBUF
E_PAD = NCHUNKS_P * CHUNK
N_PAD = 10240
ROWS_PER_TILE = N_PAD // NS
ZBLK = 64

_mesh = plsc.VectorSubcoreMesh(core_axis_name="c", subcore_axis_name="s")


@functools.partial(
    pl.kernel,
    out_type=jax.ShapeDtypeStruct((NC, N_PAD, LANES), jnp.float32),
    mesh=_mesh,
    scratch_types=[
        pltpu.VMEM_SHARED((N_PAD, LANES), jnp.float32),
        pltpu.VMEM((NBUF, 2, CHUNK), jnp.int32),
        pltpu.VMEM((NBUF, CHUNK, LANES), jnp.float32),
        pltpu.SemaphoreType.DMA((NBUF,)),
        pltpu.SemaphoreType.DMA((NBUF,)),
    ],
)
def _sc_degree(idx2_hbm, ewb_hbm, out_hbm, acc_sh, idx_v, ewb_v, sem_io, sem_s):
    c = lax.axis_index("c")
    s = lax.axis_index("s")
    wid = s * NC + c
    row0 = s * ROWS_PER_TILE
    start = wid * NCHT_D

    def prefetch(b, q):
        pltpu.async_copy(idx2_hbm.at[q], idx_v.at[b], sem_io.at[b])
        pltpu.async_copy(ewb_hbm.at[q], ewb_v.at[b], sem_io.at[b])

    def wait_prefetch(b, q):
        pltpu.make_async_copy(idx2_hbm.at[q], idx_v.at[b], sem_io.at[b]).wait()
        pltpu.make_async_copy(ewb_hbm.at[q], ewb_v.at[b], sem_io.at[b]).wait()

    def scatter(b):
        pltpu.async_copy(ewb_v.at[b], acc_sh.at[idx_v.at[b, 1]], sem_s.at[b],
                         add=True)

    def wait_scatter(b):
        pltpu.make_async_copy(ewb_v.at[b], acc_sh.at[idx_v.at[b, 1]],
                              sem_s.at[b]).wait()

    @pl.loop(0, CHUNK)
    def _(r):
        ewb_v[0, r, :] = jnp.zeros((LANES,), jnp.float32)

    @pl.loop(0, ROWS_PER_TILE // ZBLK)
    def _(i):
        pltpu.sync_copy(ewb_v.at[0],
                        acc_sh.at[pl.ds(row0 + i * ZBLK, ZBLK)])

    plsc.subcore_barrier()

    prefetch(0, start)
    prefetch(1, start + 1)
    for b in range(NBUF):
        wait_prefetch(b, start + b)
        scatter(b)
        if b >= 2:
            wait_scatter(b - 2)
        prefetch((b + 2) % NBUF, start + b + 2)

    @pl.loop(1, NCHT_D // NBUF)
    def _(i):
        base = start + i * NBUF
        for b in range(NBUF):
            q = base + b
            wait_prefetch(b, q)
            scatter(b)
            m = (b + 2) % NBUF
            wait_scatter(m)
            prefetch(m, q + 2)

    for b in range(NBUF):
        if b < 2:
            wait_prefetch(b, start + NCHT_D + b)
        else:
            wait_scatter(b)

    plsc.subcore_barrier()

    @pl.loop(0, ROWS_PER_TILE // ZBLK)
    def _(i):
        r = row0 + i * ZBLK
        pltpu.sync_copy(acc_sh.at[pl.ds(r, ZBLK)], ewb_v.at[0])
        pltpu.sync_copy(ewb_v.at[0], out_hbm.at[c, pl.ds(r, ZBLK)])


@functools.partial(
    pl.kernel,
    out_type=jax.ShapeDtypeStruct((NC, N_PAD, DH), jnp.float32),
    mesh=_mesh,
    scratch_types=[
        pltpu.VMEM_SHARED((N_PAD, DH), jnp.float32),
        pltpu.VMEM_SHARED((N_PAD, DH), jnp.float32),
        pltpu.VMEM((NBUF, 2, CHUNK), jnp.int32),
        pltpu.VMEM((NBUF, CHUNK * LANES), jnp.float32),
        pltpu.VMEM((NBUF, CHUNK, DH), jnp.float32),
        pltpu.SemaphoreType.DMA((NBUF,)),
        pltpu.SemaphoreType.DMA((NBUF,)),
        pltpu.SemaphoreType.DMA((NBUF,)),
    ],
)
def _sc_aggregate(idx2_hbm, ewb_hbm, hs_hbm, out_hbm,
                  acc_sh, hs_sh, idx_v, ewb_v, rows_v, sem_io, sem_g, sem_s):
    c = lax.axis_index("c")
    s = lax.axis_index("s")
    row0 = s * ROWS_PER_TILE
    start = s * NCHT_A

    def prefetch(b, q):
        pltpu.async_copy(idx2_hbm.at[q], idx_v.at[b], sem_io.at[b])
        pltpu.async_copy(ewb_hbm.at[q], ewb_v.at[b], sem_io.at[b])

    def wait_prefetch(b, q):
        pltpu.make_async_copy(idx2_hbm.at[q], idx_v.at[b], sem_io.at[b]).wait()
        pltpu.make_async_copy(ewb_hbm.at[q], ewb_v.at[b], sem_io.at[b]).wait()

    def launch_gather(b, q):
        wait_prefetch(b, q)
        pltpu.async_copy(hs_sh.at[idx_v.at[b, 0]], rows_v.at[b], sem_g.at[b])

    def process(b):
        pltpu.make_async_copy(hs_sh.at[idx_v.at[b, 0]], rows_v.at[b],
                              sem_g.at[b]).wait()

        @pl.loop(0, CHUNK)
        def _(e):
            w = ewb_v[b, pl.ds(e * LANES, LANES)]
            for k in range(DH // LANES):
                sl = pl.ds(k * LANES, LANES)
                rows_v[b, e, sl] = rows_v[b, e, sl] * w

        pltpu.async_copy(rows_v.at[b], acc_sh.at[idx_v.at[b, 1]], sem_s.at[b],
                         add=True)

    def wait_scatter(b):
        pltpu.make_async_copy(rows_v.at[b], acc_sh.at[idx_v.at[b, 1]],
                              sem_s.at[b]).wait()

    @pl.loop(0, CHUNK)
    def _(r):
        for k in range(DH // LANES):
            rows_v[0, r, pl.ds(k * LANES, LANES)] = jnp.zeros((LANES,),
                                                              jnp.float32)

    @pl.loop(0, ROWS_PER_TILE // ZBLK)
    def _(i):
        pltpu.sync_copy(rows_v.at[0],
                        acc_sh.at[pl.ds(row0 + i * ZBLK, ZBLK)])

    @pl.loop(0, ROWS_PER_TILE // ZBLK)
    def _(i):
        r = row0 + i * ZBLK
        pltpu.sync_copy(hs_hbm.at[c, pl.ds(r, ZBLK)], rows_v.at[0])
        pltpu.sync_copy(rows_v.at[0], hs_sh.at[pl.ds(r, ZBLK)])

    plsc.subcore_barrier()

    prefetch(0, start)
    prefetch(1, start + 1)
    prefetch(2, start + 2)
    launch_gather(0, start)
    launch_gather(1, start + 1)
    for b in range(NBUF):
        process(b)
        mb = (b + 3) % NBUF
        if b >= 1:
            wait_scatter(mb)
        prefetch(mb, start + b + 3)
        launch_gather((b + 2) % NBUF, start + b + 2)

    @pl.loop(1, NCHT_A // NBUF)
    def _(i):
        base = start + i * NBUF
        for b in range(NBUF):
            q = base + b
            process(b)
            mb = (b + 3) % NBUF
            wait_scatter(mb)
            prefetch(mb, q + 3)
            launch_gather((b + 2) % NBUF, q + 2)

    wait_scatter(3)
    pltpu.make_async_copy(hs_sh.at[idx_v.at[0, 0]], rows_v.at[0],
                          sem_g.at[0]).wait()
    pltpu.make_async_copy(hs_sh.at[idx_v.at[1, 0]], rows_v.at[1],
                          sem_g.at[1]).wait()
    wait_prefetch(2, start + NCHT_A + 2)

    plsc.subcore_barrier()

    @pl.loop(0, ROWS_PER_TILE // ZBLK)
    def _(i):
        r = row0 + i * ZBLK
        pltpu.sync_copy(acc_sh.at[pl.ds(r, ZBLK)], rows_v.at[0])
        pltpu.sync_copy(rows_v.at[0], out_hbm.at[c, pl.ds(r, ZBLK)])


MBLK = 640
EBLK = 20000


def _ewb_body(ew_ref, out_ref):
    out_ref[...] = jnp.broadcast_to(ew_ref[...], (EBLK, LANES))


def _ew_broadcast(ew):
    return pl.pallas_call(
        _ewb_body,
        grid=(E // EBLK,),
        in_specs=[pl.BlockSpec((EBLK, 1), lambda i: (i, 0))],
        out_specs=pl.BlockSpec((EBLK, LANES), lambda i: (i, 0)),
        out_shape=jax.ShapeDtypeStruct((E, LANES), jnp.float32),
    )(ew)


def _mm_body(x_ref, w_ref, out_ref):
    out_ref[...] = jnp.dot(x_ref[...], w_ref[...],
                           preferred_element_type=jnp.float32)


def _matmul(x, w):
    return pl.pallas_call(
        _mm_body,
        grid=(N_PAD // MBLK,),
        in_specs=[pl.BlockSpec((MBLK, D), lambda i: (i, 0)),
                  pl.BlockSpec((D, D), lambda i: (0, 0))],
        out_specs=pl.BlockSpec((MBLK, D), lambda i: (i, 0)),
        out_shape=jax.ShapeDtypeStruct((N_PAD, D), jnp.float32),
    )(x, w)


def _scale_body(dgp_ref, h_ref, hs_ref, dinv_ref):
    deg = dgp_ref[0, :, 0:1] + dgp_ref[1, :, 0:1] + 1.0
    dinv = lax.rsqrt(deg)
    dinv_ref[...] = dinv
    hs = h_ref[...] * dinv
    hs_ref[0] = hs[:, :DH]
    hs_ref[1] = hs[:, DH:]


def _scale(dgp, h):
    return pl.pallas_call(
        _scale_body,
        grid=(N_PAD // MBLK,),
        in_specs=[pl.BlockSpec((NC, MBLK, LANES), lambda i: (0, i, 0)),
                  pl.BlockSpec((MBLK, D), lambda i: (i, 0))],
        out_specs=[pl.BlockSpec((NC, MBLK, DH), lambda i: (0, i, 0)),
                   pl.BlockSpec((MBLK, 1), lambda i: (i, 0))],
        out_shape=[jax.ShapeDtypeStruct((NC, N_PAD, DH), jnp.float32),
                   jax.ShapeDtypeStruct((N_PAD, 1), jnp.float32)],
    )(dgp, h)


def _elu(t):
    return jnp.where(t > 0.0, t, jnp.exp(t) - 1.0)


def _mid_body(p_ref, hs_ref, dinv_ref, w_ref, b_ref, out_ref):
    msg = jnp.concatenate([p_ref[0], p_ref[1]], axis=1)
    hs = jnp.concatenate([hs_ref[0], hs_ref[1]], axis=1)
    t = (msg + hs) * dinv_ref[...] + b_ref[...]
    h2 = jnp.dot(_elu(t), w_ref[...],
                 preferred_element_type=jnp.float32) * dinv_ref[...]
    out_ref[0] = h2[:, :DH]
    out_ref[1] = h2[:, DH:]


def _mid(p, hs, dinv, w, b):
    return pl.pallas_call(
        _mid_body,
        grid=(N_PAD // MBLK,),
        in_specs=[pl.BlockSpec((NC, MBLK, DH), lambda i: (0, i, 0)),
                  pl.BlockSpec((NC, MBLK, DH), lambda i: (0, i, 0)),
                  pl.BlockSpec((MBLK, 1), lambda i: (i, 0)),
                  pl.BlockSpec((D, D), lambda i: (0, 0)),
                  pl.BlockSpec((1, D), lambda i: (0, 0))],
        out_specs=pl.BlockSpec((NC, MBLK, DH), lambda i: (0, i, 0)),
        out_shape=jax.ShapeDtypeStruct((NC, N_PAD, DH), jnp.float32),
    )(p, hs, dinv, w, b)


def _final_body(q_ref, hs_ref, dinv_ref, b_ref, wl_ref, bl_ref, out_ref):
    msg = jnp.concatenate([q_ref[0], q_ref[1]], axis=1)
    hs = jnp.concatenate([hs_ref[0], hs_ref[1]], axis=1)
    t = (msg + hs) * dinv_ref[...] + b_ref[...]
    a = _elu(t)
    out_ref[...] = jnp.sum(a * wl_ref[...], axis=1, keepdims=True) + bl_ref[...]


def _final(q, hs, dinv, b, wlin_t, blin):
    return pl.pallas_call(
        _final_body,
        grid=(N_PAD // MBLK,),
        in_specs=[pl.BlockSpec((NC, MBLK, DH), lambda i: (0, i, 0)),
                  pl.BlockSpec((NC, MBLK, DH), lambda i: (0, i, 0)),
                  pl.BlockSpec((MBLK, 1), lambda i: (i, 0)),
                  pl.BlockSpec((1, D), lambda i: (0, 0)),
                  pl.BlockSpec((1, D), lambda i: (0, 0)),
                  pl.BlockSpec((1, 1), lambda i: (0, 0))],
        out_specs=pl.BlockSpec((MBLK, 1), lambda i: (i, 0)),
        out_shape=jax.ShapeDtypeStruct((N_PAD, 1), jnp.float32),
    )(q, hs, dinv, b, wlin_t, blin)


def kernel(x, edge_index, weights_matrix, W1, b1, W2, b2, Wlin, blin):
    pad = E_PAD - E
    src_p = jnp.concatenate([edge_index[0],
                             jnp.zeros((pad,), jnp.int32)])
    dst_p = jnp.concatenate([edge_index[1],
                             jnp.full((pad,), N, jnp.int32)])
    idx2 = (jnp.stack([src_p, dst_p], axis=0)
            .reshape(2, NCHUNKS_P, CHUNK).transpose(1, 0, 2))

    ewb = _ew_broadcast(weights_matrix.reshape(E, 1))
    ewb = jnp.concatenate([ewb, jnp.zeros((pad, LANES), jnp.float32)])
    ewb3 = ewb.reshape(NCHUNKS_P, CHUNK, LANES)
    ewbf = ewb.reshape(NCHUNKS_P, CHUNK * LANES)

    xp = jnp.pad(x, ((0, N_PAD - N), (0, 0)))

    dgp = _sc_degree(idx2, ewb3)
    h1 = _matmul(xp, W1)

    hs1, dinv = _scale(dgp, h1)
    p = _sc_aggregate(idx2, ewbf, hs1)
    hs2 = _mid(p, hs1, dinv, W2, b1.reshape(1, D))
    q = _sc_aggregate(idx2, ewbf, hs2)
    out = _final(q, hs2, dinv, b2.reshape(1, D), Wlin.reshape(1, D),
                 blin.reshape(1, 1))
    return out.reshape(N_PAD)[:N]


# speedup vs baseline: 2.4843x; 1.6764x over previous
import dataclasses
import functools

import jax
import jax.numpy as jnp
from jax import lax
from jax.experimental import pallas as pl
from jax.experimental.pallas import tpu as pltpu
from jax.experimental.pallas import tpu_sc as plsc

N = 10000
E = 320000
D = 128
DH = D // 2
NC, NS, LANES = 2, 16, 16
NW = NC * NS
CHUNK = 64
NBUF = 4
NCHT_D = 160
NCHT_A = 320
NCHUNKS_P = NW * NCHT_D + NBUF
E_PAD = NCHUNKS_P * CHUNK
N_PAD = 10240
ROWS_PER_TILE = N_PAD // NS
ZBLK = 64

_mesh = plsc.VectorSubcoreMesh(core_axis_name="c", subcore_axis_name="s")

_sc_params = pltpu.CompilerParams()
if "needs_layout_passes" in pltpu.CompilerParams.__dataclass_fields__:
    _sc_params = dataclasses.replace(_sc_params, needs_layout_passes=False)


@functools.partial(
    pl.kernel,
    out_type=jax.ShapeDtypeStruct((NC, N_PAD, LANES), jnp.float32),
    mesh=_mesh,
    scratch_types=[
        pltpu.VMEM_SHARED((N_PAD, LANES), jnp.float32),
        pltpu.VMEM((NBUF, 2, CHUNK), jnp.int32),
        pltpu.VMEM((NBUF, CHUNK), jnp.float32),
        pltpu.VMEM((NBUF, CHUNK, LANES), jnp.float32),
        pltpu.SemaphoreType.DMA((NBUF,)),
        pltpu.SemaphoreType.DMA((NBUF,)),
    ],
    compiler_params=_sc_params,
)
def _sc_degree(idx2_hbm, ew_hbm, out_hbm, acc_sh, idx_v, ew_v, rows_v,
               sem_io, sem_s):
    c = lax.axis_index("c")
    s = lax.axis_index("s")
    wid = s * NC + c
    row0 = s * ROWS_PER_TILE
    start = wid * NCHT_D

    def prefetch(b, q):
        pltpu.async_copy(idx2_hbm.at[q], idx_v.at[b], sem_io.at[b])
        pltpu.async_copy(ew_hbm.at[q], ew_v.at[b], sem_io.at[b])

    def wait_prefetch(b, q):
        pltpu.make_async_copy(idx2_hbm.at[q], idx_v.at[b], sem_io.at[b]).wait()
        pltpu.make_async_copy(ew_hbm.at[q], ew_v.at[b], sem_io.at[b]).wait()

    def scatter(b):
        @pl.loop(0, CHUNK)
        def _(e):
            idx = jnp.full((LANES,), e, jnp.int32)
            rows_v[b, e, :] = plsc.load_gather(ew_v.at[b], [idx])

        pltpu.async_copy(rows_v.at[b], acc_sh.at[idx_v.at[b, 1]], sem_s.at[b],
                         add=True)

    def wait_scatter(b):
        pltpu.make_async_copy(rows_v.at[b], acc_sh.at[idx_v.at[b, 1]],
                              sem_s.at[b]).wait()

    @pl.loop(0, CHUNK)
    def _(r):
        rows_v[0, r, :] = jnp.zeros((LANES,), jnp.float32)

    @pl.loop(0, ROWS_PER_TILE // ZBLK)
    def _(i):
        pltpu.sync_copy(rows_v.at[0],
                        acc_sh.at[pl.ds(row0 + i * ZBLK, ZBLK)])

    plsc.subcore_barrier()

    prefetch(0, start)
    prefetch(1, start + 1)
    for b in range(NBUF):
        wait_prefetch(b, start + b)
        scatter(b)
        if b >= 2:
            wait_scatter(b - 2)
        prefetch((b + 2) % NBUF, start + b + 2)

    @pl.loop(1, NCHT_D // NBUF)
    def _(i):
        base = start + i * NBUF
        for b in range(NBUF):
            q = base + b
            wait_prefetch(b, q)
            scatter(b)
            m = (b + 2) % NBUF
            wait_scatter(m)
            prefetch(m, q + 2)

    for b in range(NBUF):
        if b < 2:
            wait_prefetch(b, start + NCHT_D + b)
        else:
            wait_scatter(b)

    plsc.subcore_barrier()

    @pl.loop(0, ROWS_PER_TILE // ZBLK)
    def _(i):
        r = row0 + i * ZBLK
        pltpu.sync_copy(acc_sh.at[pl.ds(r, ZBLK)], rows_v.at[0])
        pltpu.sync_copy(rows_v.at[0], out_hbm.at[c, pl.ds(r, ZBLK)])


@functools.partial(
    pl.kernel,
    out_type=jax.ShapeDtypeStruct((NC, N_PAD, DH), jnp.float32),
    mesh=_mesh,
    scratch_types=[
        pltpu.VMEM_SHARED((N_PAD, DH), jnp.float32),
        pltpu.VMEM_SHARED((N_PAD, DH), jnp.float32),
        pltpu.VMEM((NBUF, 2, CHUNK), jnp.int32),
        pltpu.VMEM((NBUF, CHUNK), jnp.float32),
        pltpu.VMEM((NBUF, CHUNK, DH), jnp.float32),
        pltpu.SemaphoreType.DMA((NBUF,)),
        pltpu.SemaphoreType.DMA((NBUF,)),
        pltpu.SemaphoreType.DMA((NBUF,)),
    ],
    compiler_params=_sc_params,
)
def _sc_aggregate(idx2_hbm, ew_hbm, hs_hbm, out_hbm,
                  acc_sh, hs_sh, idx_v, ew_v, rows_v,
                  sem_io, sem_g, sem_s):
    c = lax.axis_index("c")
    s = lax.axis_index("s")
    row0 = s * ROWS_PER_TILE
    start = s * NCHT_A

    def prefetch(b, q):
        pltpu.async_copy(idx2_hbm.at[q], idx_v.at[b], sem_io.at[b])
        pltpu.async_copy(ew_hbm.at[q], ew_v.at[b], sem_io.at[b])

    def wait_prefetch(b, q):
        pltpu.make_async_copy(idx2_hbm.at[q], idx_v.at[b], sem_io.at[b]).wait()
        pltpu.make_async_copy(ew_hbm.at[q], ew_v.at[b], sem_io.at[b]).wait()

    def launch_gather(b, q):
        wait_prefetch(b, q)
        pltpu.async_copy(hs_sh.at[idx_v.at[b, 0]], rows_v.at[b], sem_g.at[b])

    def process(b):
        pltpu.make_async_copy(hs_sh.at[idx_v.at[b, 0]], rows_v.at[b],
                              sem_g.at[b]).wait()

        @pl.loop(0, CHUNK)
        def _(e):
            w = plsc.load_gather(ew_v.at[b], [jnp.full((LANES,), e, jnp.int32)])
            for k in range(DH // LANES):
                sl = pl.ds(k * LANES, LANES)
                rows_v[b, e, sl] = rows_v[b, e, sl] * w

        pltpu.async_copy(rows_v.at[b], acc_sh.at[idx_v.at[b, 1]], sem_s.at[b],
                         add=True)

    def wait_scatter(b):
        pltpu.make_async_copy(rows_v.at[b], acc_sh.at[idx_v.at[b, 1]],
                              sem_s.at[b]).wait()

    @pl.loop(0, CHUNK)
    def _(r):
        for k in range(DH // LANES):
            rows_v[0, r, pl.ds(k * LANES, LANES)] = jnp.zeros((LANES,),
                                                              jnp.float32)

    @pl.loop(0, ROWS_PER_TILE // ZBLK)
    def _(i):
        pltpu.sync_copy(rows_v.at[0],
                        acc_sh.at[pl.ds(row0 + i * ZBLK, ZBLK)])

    @pl.loop(0, ROWS_PER_TILE // ZBLK)
    def _(i):
        r = row0 + i * ZBLK
        pltpu.sync_copy(hs_hbm.at[c, pl.ds(r, ZBLK)], rows_v.at[0])
        pltpu.sync_copy(rows_v.at[0], hs_sh.at[pl.ds(r, ZBLK)])

    plsc.subcore_barrier()

    prefetch(0, start)
    prefetch(1, start + 1)
    prefetch(2, start + 2)
    launch_gather(0, start)
    launch_gather(1, start + 1)
    for b in range(NBUF):
        process(b)
        mb = (b + 3) % NBUF
        if b >= 1:
            wait_scatter(mb)
        prefetch(mb, start + b + 3)
        launch_gather((b + 2) % NBUF, start + b + 2)

    @pl.loop(1, NCHT_A // NBUF)
    def _(i):
        base = start + i * NBUF
        for b in range(NBUF):
            q = base + b
            process(b)
            mb = (b + 3) % NBUF
            wait_scatter(mb)
            prefetch(mb, q + 3)
            launch_gather((b + 2) % NBUF, q + 2)

    wait_scatter(3)
    pltpu.make_async_copy(hs_sh.at[idx_v.at[0, 0]], rows_v.at[0],
                          sem_g.at[0]).wait()
    pltpu.make_async_copy(hs_sh.at[idx_v.at[1, 0]], rows_v.at[1],
                          sem_g.at[1]).wait()
    wait_prefetch(2, start + NCHT_A + 2)

    plsc.subcore_barrier()

    @pl.loop(0, ROWS_PER_TILE // ZBLK)
    def _(i):
        r = row0 + i * ZBLK
        pltpu.sync_copy(acc_sh.at[pl.ds(r, ZBLK)], rows_v.at[0])
        pltpu.sync_copy(rows_v.at[0], out_hbm.at[c, pl.ds(r, ZBLK)])


MBLK = 640


def _mm_body(x_ref, w_ref, out_ref):
    out_ref[...] = jnp.dot(x_ref[...], w_ref[...],
                           preferred_element_type=jnp.float32)


def _matmul(x, w):
    return pl.pallas_call(
        _mm_body,
        grid=(N_PAD // MBLK,),
        in_specs=[pl.BlockSpec((MBLK, D), lambda i: (i, 0)),
                  pl.BlockSpec((D, D), lambda i: (0, 0))],
        out_specs=pl.BlockSpec((MBLK, D), lambda i: (i, 0)),
        out_shape=jax.ShapeDtypeStruct((N_PAD, D), jnp.float32),
    )(x, w)


def _scale_body(dgp_ref, h_ref, hs_ref, dinv_ref):
    deg = dgp_ref[0, :, 0:1] + dgp_ref[1, :, 0:1] + 1.0
    dinv = lax.rsqrt(deg)
    dinv_ref[...] = dinv
    hs = h_ref[...] * dinv
    hs_ref[0] = hs[:, :DH]
    hs_ref[1] = hs[:, DH:]


def _scale(dgp, h):
    return pl.pallas_call(
        _scale_body,
        grid=(N_PAD // MBLK,),
        in_specs=[pl.BlockSpec((NC, MBLK, LANES), lambda i: (0, i, 0)),
                  pl.BlockSpec((MBLK, D), lambda i: (i, 0))],
        out_specs=[pl.BlockSpec((NC, MBLK, DH), lambda i: (0, i, 0)),
                   pl.BlockSpec((MBLK, 1), lambda i: (i, 0))],
        out_shape=[jax.ShapeDtypeStruct((NC, N_PAD, DH), jnp.float32),
                   jax.ShapeDtypeStruct((N_PAD, 1), jnp.float32)],
    )(dgp, h)


def _elu(t):
    return jnp.where(t > 0.0, t, jnp.exp(t) - 1.0)


def _mid_body(p_ref, hs_ref, dinv_ref, w_ref, b_ref, out_ref):
    msg = jnp.concatenate([p_ref[0], p_ref[1]], axis=1)
    hs = jnp.concatenate([hs_ref[0], hs_ref[1]], axis=1)
    t = (msg + hs) * dinv_ref[...] + b_ref[...]
    h2 = jnp.dot(_elu(t), w_ref[...],
                 preferred_element_type=jnp.float32) * dinv_ref[...]
    out_ref[0] = h2[:, :DH]
    out_ref[1] = h2[:, DH:]


def _mid(p, hs, dinv, w, b):
    return pl.pallas_call(
        _mid_body,
        grid=(N_PAD // MBLK,),
        in_specs=[pl.BlockSpec((NC, MBLK, DH), lambda i: (0, i, 0)),
                  pl.BlockSpec((NC, MBLK, DH), lambda i: (0, i, 0)),
                  pl.BlockSpec((MBLK, 1), lambda i: (i, 0)),
                  pl.BlockSpec((D, D), lambda i: (0, 0)),
                  pl.BlockSpec((1, D), lambda i: (0, 0))],
        out_specs=pl.BlockSpec((NC, MBLK, DH), lambda i: (0, i, 0)),
        out_shape=jax.ShapeDtypeStruct((NC, N_PAD, DH), jnp.float32),
    )(p, hs, dinv, w, b)


def _final_body(q_ref, hs_ref, dinv_ref, b_ref, wl_ref, bl_ref, out_ref):
    msg = jnp.concatenate([q_ref[0], q_ref[1]], axis=1)
    hs = jnp.concatenate([hs_ref[0], hs_ref[1]], axis=1)
    t = (msg + hs) * dinv_ref[...] + b_ref[...]
    a = _elu(t)
    out_ref[...] = jnp.sum(a * wl_ref[...], axis=1, keepdims=True) + bl_ref[...]


def _final(q, hs, dinv, b, wlin_t, blin):
    return pl.pallas_call(
        _final_body,
        grid=(N_PAD // MBLK,),
        in_specs=[pl.BlockSpec((NC, MBLK, DH), lambda i: (0, i, 0)),
                  pl.BlockSpec((NC, MBLK, DH), lambda i: (0, i, 0)),
                  pl.BlockSpec((MBLK, 1), lambda i: (i, 0)),
                  pl.BlockSpec((1, D), lambda i: (0, 0)),
                  pl.BlockSpec((1, D), lambda i: (0, 0)),
                  pl.BlockSpec((1, 1), lambda i: (0, 0))],
        out_specs=pl.BlockSpec((MBLK, 1), lambda i: (i, 0)),
        out_shape=jax.ShapeDtypeStruct((N_PAD, 1), jnp.float32),
    )(q, hs, dinv, b, wlin_t, blin)


def kernel(x, edge_index, weights_matrix, W1, b1, W2, b2, Wlin, blin):
    pad = E_PAD - E
    src_p = jnp.concatenate([edge_index[0],
                             jnp.zeros((pad,), jnp.int32)])
    dst_p = jnp.concatenate([edge_index[1],
                             jnp.full((pad,), N, jnp.int32)])
    idx2 = (jnp.stack([src_p, dst_p], axis=0)
            .reshape(2, NCHUNKS_P, CHUNK).transpose(1, 0, 2))

    ewp = jnp.pad(weights_matrix, (0, pad)).reshape(NCHUNKS_P, CHUNK)

    dgp = _sc_degree(idx2, ewp)
    h1 = _matmul(x, W1)

    hs1, dinv = _scale(dgp, h1)
    p = _sc_aggregate(idx2, ewp, hs1)
    hs2 = _mid(p, hs1, dinv, W2, b1.reshape(1, D))
    q = _sc_aggregate(idx2, ewp, hs2)
    out = _final(q, hs2, dinv, b2.reshape(1, D), Wlin.reshape(1, D),
                 blin.reshape(1, 1))
    return out.reshape(N_PAD)[:N]
